# baseline (speedup 1.0000x reference)
import functools

import jax
import jax.numpy as jnp
from jax import lax
from jax.experimental import pallas as pl
from jax.experimental.pallas import tpu as pltpu
from jax.experimental.pallas import tpu_sc as plsc

EMBED_DIM = 32
PRED_DIM = 128
BATCH = 16384
NUM_EMB = 1000000

_info = plsc.get_sparse_core_info()
_NC, _NS = _info.num_cores, _info.num_subcores
_NW = _NC * _NS
_BPW = BATCH // _NW

_mesh = plsc.VectorSubcoreMesh(core_axis_name="c", subcore_axis_name="s")


_SC_COLS = 393216
_CH = 1024
_CPS = _SC_COLS // _NW
_NCHUNK = _CPS // _CH
_TC_COLS = NUM_EMB - _SC_COLS


@functools.partial(
    pl.kernel,
    mesh=_mesh,
    out_type=jax.ShapeDtypeStruct((_SC_COLS,), jnp.float32),
    scratch_types=[
        pltpu.VMEM((2, EMBED_DIM, _CH), jnp.float32),
        pltpu.VMEM((2, _CH), jnp.float32),
        pltpu.VMEM((EMBED_DIM,), jnp.float32),
        pltpu.SemaphoreType.DMA,
        pltpu.SemaphoreType.DMA,
        pltpu.SemaphoreType.DMA,
        pltpu.SemaphoreType.DMA,
    ],
)
def _sc_table_dot(tbl_hbm, we_hbm, out_hbm, v_buf, o_buf, we_v,
                  si0, si1, so0, so1):
    wid = lax.axis_index("s") * _NC + lax.axis_index("c")
    base = wid * _CPS
    pltpu.sync_copy(we_hbm, we_v)
    wv = [we_v[pl.ds(0, 16)], we_v[pl.ds(16, 16)]]
    ws = [wv[j // 16][j % 16] for j in range(EMBED_DIM)]
    sin = (si0, si1)
    sout = (so0, so1)
    in_h = [None, None]
    out_h = [None, None]

    def start_in(c):
        b = c % 2
        in_h[b] = pltpu.async_copy(
            tbl_hbm.at[:, pl.ds(base + c * _CH, _CH)], v_buf.at[b], sin[b])

    start_in(0)
    if _NCHUNK > 1:
        start_in(1)
    for c in range(_NCHUNK):
        b = c % 2
        in_h[b].wait()
        if out_h[b] is not None:
            out_h[b].wait()
        v = v_buf.at[b]
        o = o_buf.at[b]

        def body(g, _, v=v, o=o):
            sl0 = pl.ds(g * 32, 16)
            sl1 = pl.ds(g * 32 + 16, 16)
            acc0 = v[0, sl0] * ws[0]
            acc1 = v[0, sl1] * ws[0]
            for j in range(1, EMBED_DIM):
                acc0 = acc0 + v[j, sl0] * ws[j]
                acc1 = acc1 + v[j, sl1] * ws[j]
            o[sl0] = acc0
            o[sl1] = acc1
            return 0

        lax.fori_loop(0, _CH // 32, body, 0)
        out_h[b] = pltpu.async_copy(
            o, out_hbm.at[pl.ds(base + c * _CH, _CH)], sout[b])
        if c + 2 < _NCHUNK:
            start_in(c + 2)
    out_h[0].wait()
    if out_h[1] is not None:
        out_h[1].wait()


_DOT_BLK = 65536


def _table_dot_body(tbl_ref, we_ref, out_ref):
    out_ref[...] = jnp.sum(tbl_ref[...] * we_ref[...], axis=0)


def _table_dot_tc(table_t, W):
    we = W[PRED_DIM:, :]
    off = _SC_COLS // _DOT_BLK
    grid = (pl.cdiv(_TC_COLS, _DOT_BLK),)
    return pl.pallas_call(
        _table_dot_body,
        grid=grid,
        in_specs=[
            pl.BlockSpec((EMBED_DIM, _DOT_BLK), lambda i: (0, off + i)),
            pl.BlockSpec((EMBED_DIM, 1), lambda i: (0, 0)),
        ],
        out_specs=pl.BlockSpec((_DOT_BLK,), lambda i: (i,)),
        out_shape=jax.ShapeDtypeStruct((_TC_COLS,), jnp.float32),
    )(table_t, we)


_ROWS_BLK = 4096


def _pred_body(pred_ref, w_ref, b_ref, out_ref):
    acc = jnp.dot(pred_ref[...], w_ref[...], preferred_element_type=jnp.float32)
    out_ref[...] = (acc + b_ref[...])[:, 0]


def _pred_term(predictors, W, b):
    wp = W[:PRED_DIM, :]
    grid = (BATCH // _ROWS_BLK,)
    return pl.pallas_call(
        _pred_body,
        grid=grid,
        in_specs=[
            pl.BlockSpec((_ROWS_BLK, PRED_DIM), lambda i: (i, 0)),
            pl.BlockSpec((PRED_DIM, 1), lambda i: (0, 0)),
            pl.BlockSpec((1,), lambda i: (0,)),
        ],
        out_specs=pl.BlockSpec((_ROWS_BLK,), lambda i: (i,)),
        out_shape=jax.ShapeDtypeStruct((BATCH,), jnp.float32),
    )(predictors, wp, b)


@functools.partial(
    pl.kernel,
    mesh=_mesh,
    out_type=jax.ShapeDtypeStruct((BATCH,), jnp.float32),
    compiler_params=pltpu.CompilerParams(use_tc_tiling_on_sc=False),
    scratch_types=[
        pltpu.VMEM((_BPW,), jnp.int32),
        pltpu.VMEM((_BPW,), jnp.int32),
        pltpu.VMEM((_BPW,), jnp.int32),
        pltpu.VMEM((_BPW,), jnp.float32),
        pltpu.VMEM((_BPW,), jnp.float32),
        pltpu.VMEM((_BPW,), jnp.float32),
        pltpu.SemaphoreType.DMA,
        pltpu.SemaphoreType.DMA,
        pltpu.SemaphoreType.DMA,
    ],
)
def _sc_gather_add(dot_sc_hbm, dot_tc_hbm, pred_hbm, idx_hbm, out_hbm,
                   idx_v, ia_v, ib_v, va_v, vb_v, pt_v, s0, s1, s2):
    wid = lax.axis_index("s") * _NC + lax.axis_index("c")
    base = wid * _BPW
    pltpu.sync_copy(idx_hbm.at[pl.ds(base, _BPW)], idx_v)
    hp = pltpu.async_copy(pred_hbm.at[pl.ds(base, _BPW)], pt_v, s2)

    def split(g, _):
        sl = pl.ds(g * 16, 16)
        iv = idx_v[sl]
        ib_v[sl] = jnp.minimum(iv, _SC_COLS - 1)
        ia_v[sl] = jnp.minimum(jnp.maximum(iv - _SC_COLS, 0), _TC_COLS - 1)
        return 0

    lax.fori_loop(0, _BPW // 16, split, 0)
    ha = pltpu.async_copy(dot_tc_hbm.at[ia_v], va_v, s0)
    hb = pltpu.async_copy(dot_sc_hbm.at[ib_v], vb_v, s1)
    ha.wait()
    hb.wait()
    hp.wait()

    def merge(g, _):
        sl = pl.ds(g * 16, 16)
        sel = idx_v[sl] < _SC_COLS
        vb_v[sl] = jnp.where(sel, vb_v[sl], va_v[sl]) + pt_v[sl]
        return 0

    lax.fori_loop(0, _BPW // 16, merge, 0)
    pltpu.sync_copy(vb_v, out_hbm.at[pl.ds(base, _BPW)])


def kernel(predictors, encoding, emb_table, W, b):
    table_t = emb_table.T
    dot_sc = _sc_table_dot(table_t, W[PRED_DIM:, 0])
    dot_tc = _table_dot_tc(table_t, W)
    pred_t = _pred_term(predictors, W, b)
    out = _sc_gather_add(dot_sc, dot_tc, pred_t, encoding)
    return out.reshape(BATCH, 1)


# speedup vs baseline: 1.3266x; 1.3266x over previous
import functools

import jax
import jax.numpy as jnp
from jax import lax
from jax.experimental import pallas as pl
from jax.experimental.pallas import tpu as pltpu
from jax.experimental.pallas import tpu_sc as plsc

EMBED_DIM = 32
PRED_DIM = 128
BATCH = 16384
NUM_EMB = 1000000

_info = plsc.get_sparse_core_info()
_NC, _NS = _info.num_cores, _info.num_subcores
_NW = _NC * _NS
_BPW = BATCH // _NW

_mesh = plsc.VectorSubcoreMesh(core_axis_name="c", subcore_axis_name="s")


_SC_COLS = 458752
_CH = 1024
_CPS = _SC_COLS // _NW
_NCHUNK = _CPS // _CH
_TC_COLS = NUM_EMB - _SC_COLS


@functools.partial(
    pl.kernel,
    mesh=_mesh,
    out_type=jax.ShapeDtypeStruct((_SC_COLS,), jnp.float32),
    scratch_types=[
        pltpu.VMEM((2, EMBED_DIM, _CH), jnp.float32),
        pltpu.VMEM((2, _CH), jnp.float32),
        pltpu.VMEM((EMBED_DIM,), jnp.float32),
        pltpu.SemaphoreType.DMA,
        pltpu.SemaphoreType.DMA,
        pltpu.SemaphoreType.DMA,
        pltpu.SemaphoreType.DMA,
    ],
)
def _sc_table_dot(tbl_hbm, we_hbm, out_hbm, v_buf, o_buf, we_v,
                  si0, si1, so0, so1):
    wid = lax.axis_index("s") * _NC + lax.axis_index("c")
    base = wid * _CPS
    pltpu.sync_copy(we_hbm, we_v)
    wv = [we_v[pl.ds(0, 16)], we_v[pl.ds(16, 16)]]
    ws = [wv[j // 16][j % 16] for j in range(EMBED_DIM)]
    sin = (si0, si1)
    sout = (so0, so1)
    in_h = [None, None]
    out_h = [None, None]

    def start_in(c):
        b = c % 2
        in_h[b] = pltpu.async_copy(
            tbl_hbm.at[:, pl.ds(base + c * _CH, _CH)], v_buf.at[b], sin[b])

    start_in(0)
    if _NCHUNK > 1:
        start_in(1)
    for c in range(_NCHUNK):
        b = c % 2
        in_h[b].wait()
        if out_h[b] is not None:
            out_h[b].wait()
        v = v_buf.at[b]
        o = o_buf.at[b]

        def body(g, _, v=v, o=o):
            sl0 = pl.ds(g * 32, 16)
            sl1 = pl.ds(g * 32 + 16, 16)
            acc0 = v[0, sl0] * ws[0]
            acc1 = v[0, sl1] * ws[0]
            for j in range(1, EMBED_DIM):
                acc0 = acc0 + v[j, sl0] * ws[j]
                acc1 = acc1 + v[j, sl1] * ws[j]
            o[sl0] = acc0
            o[sl1] = acc1
            return 0

        lax.fori_loop(0, _CH // 32, body, 0)
        out_h[b] = pltpu.async_copy(
            o, out_hbm.at[pl.ds(base + c * _CH, _CH)], sout[b])
        if c + 2 < _NCHUNK:
            start_in(c + 2)
    out_h[0].wait()
    if out_h[1] is not None:
        out_h[1].wait()


_DOT_BLK = 65536


def _table_dot_body(tbl_ref, we_ref, out_ref):
    out_ref[...] = jnp.sum(tbl_ref[...] * we_ref[...], axis=0)


def _table_dot_tc(table_t, W):
    we = W[PRED_DIM:, :]
    off = _SC_COLS // _DOT_BLK
    grid = (pl.cdiv(_TC_COLS, _DOT_BLK),)
    return pl.pallas_call(
        _table_dot_body,
        grid=grid,
        in_specs=[
            pl.BlockSpec((EMBED_DIM, _DOT_BLK), lambda i: (0, off + i)),
            pl.BlockSpec((EMBED_DIM, 1), lambda i: (0, 0)),
        ],
        out_specs=pl.BlockSpec((_DOT_BLK,), lambda i: (i,)),
        out_shape=jax.ShapeDtypeStruct((_TC_COLS,), jnp.float32),
    )(table_t, we)


_ROWS_BLK = 4096


def _pred_body(pred_ref, w_ref, b_ref, out_ref):
    acc = jnp.dot(pred_ref[...], w_ref[...], preferred_element_type=jnp.float32)
    out_ref[...] = (acc + b_ref[...])[:, 0]


def _pred_term(predictors, W, b):
    wp = W[:PRED_DIM, :]
    grid = (BATCH // _ROWS_BLK,)
    return pl.pallas_call(
        _pred_body,
        grid=grid,
        in_specs=[
            pl.BlockSpec((_ROWS_BLK, PRED_DIM), lambda i: (i, 0)),
            pl.BlockSpec((PRED_DIM, 1), lambda i: (0, 0)),
            pl.BlockSpec((1,), lambda i: (0,)),
        ],
        out_specs=pl.BlockSpec((_ROWS_BLK,), lambda i: (i,)),
        out_shape=jax.ShapeDtypeStruct((BATCH,), jnp.float32),
    )(predictors, wp, b)


@functools.partial(
    pl.kernel,
    mesh=_mesh,
    out_type=jax.ShapeDtypeStruct((BATCH,), jnp.float32),
    compiler_params=pltpu.CompilerParams(use_tc_tiling_on_sc=False),
    scratch_types=[
        pltpu.VMEM((_BPW,), jnp.int32),
        pltpu.VMEM((_BPW,), jnp.float32),
        pltpu.SemaphoreType.DMA,
    ],
)
def _sc_gather(table_dot_hbm, idx_hbm, out_hbm, idx_v, vals_v, sem):
    wid = lax.axis_index("s") * _NC + lax.axis_index("c")
    base = wid * _BPW
    pltpu.sync_copy(idx_hbm.at[pl.ds(base, _BPW)], idx_v)
    pltpu.async_copy(table_dot_hbm.at[idx_v], vals_v, sem).wait()
    pltpu.sync_copy(vals_v, out_hbm.at[pl.ds(base, _BPW)])


def _add_body(g_ref, p_ref, out_ref):
    out_ref[...] = (g_ref[...] + p_ref[...])[:, None]


def _final_add(gathered, pred_t):
    return pl.pallas_call(
        _add_body,
        out_shape=jax.ShapeDtypeStruct((BATCH, 1), jnp.float32),
    )(gathered, pred_t)


def kernel(predictors, encoding, emb_table, W, b):
    table_t = emb_table.T
    dot_sc = _sc_table_dot(table_t, W[PRED_DIM:, 0])
    dot_tc = _table_dot_tc(table_t, W)
    pred_t = _pred_term(predictors, W, b)
    table_dot = jnp.concatenate([dot_sc, dot_tc])
    gathered = _sc_gather(table_dot, encoding)
    return _final_add(gathered, pred_t)


# speedup vs baseline: 1.6615x; 1.2525x over previous
import functools

import jax
import jax.numpy as jnp
from jax import lax
from jax.experimental import pallas as pl
from jax.experimental.pallas import tpu as pltpu
from jax.experimental.pallas import tpu_sc as plsc

EMBED_DIM = 32
PRED_DIM = 128
BATCH = 16384
NUM_EMB = 1000000

_info = plsc.get_sparse_core_info()
_NC, _NS = _info.num_cores, _info.num_subcores
_NW = _NC * _NS
_BPW = BATCH // _NW

_mesh = plsc.VectorSubcoreMesh(core_axis_name="c", subcore_axis_name="s")


_SC_COLS = 458752
_CH = 1024
_CPS = _SC_COLS // _NW
_NCHUNK = _CPS // _CH
_TC_COLS = NUM_EMB - _SC_COLS


@functools.partial(
    pl.kernel,
    mesh=_mesh,
    out_type=jax.ShapeDtypeStruct((_SC_COLS,), jnp.float32),
    scratch_types=[
        pltpu.VMEM((2, EMBED_DIM, _CH), jnp.float32),
        pltpu.VMEM((2, _CH), jnp.float32),
        pltpu.VMEM((EMBED_DIM,), jnp.float32),
        pltpu.SemaphoreType.DMA,
        pltpu.SemaphoreType.DMA,
        pltpu.SemaphoreType.DMA,
        pltpu.SemaphoreType.DMA,
    ],
)
def _sc_table_dot(tbl_hbm, we_hbm, out_hbm, v_buf, o_buf, we_v,
                  si0, si1, so0, so1):
    wid = lax.axis_index("s") * _NC + lax.axis_index("c")
    base = wid * _CPS
    pltpu.sync_copy(we_hbm, we_v)
    wv = [we_v[pl.ds(0, 16)], we_v[pl.ds(16, 16)]]
    ws = [wv[j // 16][j % 16] for j in range(EMBED_DIM)]
    sin = (si0, si1)
    sout = (so0, so1)
    in_h = [None, None]
    out_h = [None, None]

    def start_in(c):
        b = c % 2
        in_h[b] = pltpu.async_copy(
            tbl_hbm.at[:, pl.ds(base + c * _CH, _CH)], v_buf.at[b], sin[b])

    start_in(0)
    if _NCHUNK > 1:
        start_in(1)
    for c in range(_NCHUNK):
        b = c % 2
        in_h[b].wait()
        if out_h[b] is not None:
            out_h[b].wait()
        v = v_buf.at[b]
        o = o_buf.at[b]

        def body(g, _, v=v, o=o):
            sl0 = pl.ds(g * 32, 16)
            sl1 = pl.ds(g * 32 + 16, 16)
            acc0 = v[0, sl0] * ws[0]
            acc1 = v[0, sl1] * ws[0]
            for j in range(1, EMBED_DIM):
                acc0 = acc0 + v[j, sl0] * ws[j]
                acc1 = acc1 + v[j, sl1] * ws[j]
            o[sl0] = acc0
            o[sl1] = acc1
            return 0

        lax.fori_loop(0, _CH // 32, body, 0)
        out_h[b] = pltpu.async_copy(
            o, out_hbm.at[pl.ds(base + c * _CH, _CH)], sout[b])
        if c + 2 < _NCHUNK:
            start_in(c + 2)
    out_h[0].wait()
    if out_h[1] is not None:
        out_h[1].wait()


_DOT_BLK = 65536


def _table_dot_body(tbl_ref, we_ref, out_ref):
    out_ref[...] = jnp.sum(tbl_ref[...] * we_ref[...], axis=0)


def _table_dot_tc(table_t, W):
    we = W[PRED_DIM:, :]
    off = _SC_COLS // _DOT_BLK
    grid = (pl.cdiv(_TC_COLS, _DOT_BLK),)
    return pl.pallas_call(
        _table_dot_body,
        grid=grid,
        in_specs=[
            pl.BlockSpec((EMBED_DIM, _DOT_BLK), lambda i: (0, off + i)),
            pl.BlockSpec((EMBED_DIM, 1), lambda i: (0, 0)),
        ],
        out_specs=pl.BlockSpec((_DOT_BLK,), lambda i: (i,)),
        out_shape=jax.ShapeDtypeStruct((_TC_COLS,), jnp.float32),
    )(table_t, we)


_ROWS_BLK = 4096


def _pred_body(pred_ref, w_ref, b_ref, out_ref):
    acc = jnp.dot(pred_ref[...], w_ref[...], preferred_element_type=jnp.float32)
    out_ref[...] = (acc + b_ref[...])[:, 0]


def _pred_term(predictors, W, b):
    wp = W[:PRED_DIM, :]
    grid = (BATCH // _ROWS_BLK,)
    return pl.pallas_call(
        _pred_body,
        grid=grid,
        in_specs=[
            pl.BlockSpec((_ROWS_BLK, PRED_DIM), lambda i: (i, 0)),
            pl.BlockSpec((PRED_DIM, 1), lambda i: (0, 0)),
            pl.BlockSpec((1,), lambda i: (0,)),
        ],
        out_specs=pl.BlockSpec((_ROWS_BLK,), lambda i: (i,)),
        out_shape=jax.ShapeDtypeStruct((BATCH,), jnp.float32),
    )(predictors, wp, b)


@functools.partial(
    pl.kernel,
    mesh=_mesh,
    out_type=jax.ShapeDtypeStruct((BATCH,), jnp.float32),
    compiler_params=pltpu.CompilerParams(use_tc_tiling_on_sc=False),
    scratch_types=[
        pltpu.VMEM((_BPW,), jnp.int32),
        pltpu.VMEM((_BPW,), jnp.float32),
        pltpu.VMEM((_BPW,), jnp.float32),
        pltpu.SemaphoreType.DMA,
        pltpu.SemaphoreType.DMA,
    ],
)
def _sc_gather(table_dot_hbm, pred_hbm, idx_hbm, out_hbm,
               idx_v, vals_v, pt_v, sem, semp):
    wid = lax.axis_index("s") * _NC + lax.axis_index("c")
    base = wid * _BPW
    pltpu.sync_copy(idx_hbm.at[pl.ds(base, _BPW)], idx_v)
    hp = pltpu.async_copy(pred_hbm.at[pl.ds(base, _BPW)], pt_v, semp)
    pltpu.async_copy(table_dot_hbm.at[idx_v], vals_v, sem).wait()
    hp.wait()

    def addp(g, _):
        sl = pl.ds(g * 16, 16)
        vals_v[sl] = vals_v[sl] + pt_v[sl]
        return 0

    lax.fori_loop(0, _BPW // 16, addp, 0)
    pltpu.sync_copy(vals_v, out_hbm.at[pl.ds(base, _BPW)])


def kernel(predictors, encoding, emb_table, W, b):
    table_t = emb_table.T
    pred_t = _pred_term(predictors, W, b)
    dot_sc = _sc_table_dot(table_t, W[PRED_DIM:, 0])
    dot_tc = _table_dot_tc(table_t, W)
    table_dot = jnp.concatenate([dot_sc, dot_tc])
    gathered = _sc_gather(table_dot, pred_t, encoding)
    return gathered.reshape(BATCH, 1)
